# baseline (speedup 1.0000x reference)
import functools

import jax
import jax.numpy as jnp
from jax import lax
from jax.experimental import pallas as pl
from jax.experimental.pallas import tpu as pltpu
from jax.experimental.pallas import tpu_sc as plsc

N_NODES = 10000
N_EDGES = 320000
D = 128

NC = 2
NS = 16
NT = NC * NS
CH = 128
NCH = -(-N_EDGES // (NT * CH))
E_PAD = NT * NCH * CH
ACC_ROWS = N_NODES + 8
ROWS_PER_TILE = 624
REM_ROWS = N_NODES - NS * ROWS_PER_TILE


def _sc_body(x_hbm, idx_hbm, zeros_hbm, out_hbm,
             ib0, ib1, rows0, rows1, acc, semi0, semi1, semr0, semr1):
    c = lax.axis_index("c")
    s = lax.axis_index("s")
    tid = c * NS + s

    pltpu.sync_copy(zeros_hbm.at[pl.ds(0, ROWS_PER_TILE)],
                    acc.at[pl.ds(s * ROWS_PER_TILE, ROWS_PER_TILE)])

    @pl.when(s == 0)
    def _():
        pltpu.sync_copy(zeros_hbm.at[pl.ds(0, REM_ROWS)],
                        acc.at[pl.ds(NS * ROWS_PER_TILE, REM_ROWS)])

    plsc.subcore_barrier()

    h = idx_hbm.at[tid]

    pltpu.async_copy(h.at[0], ib0, semi0)
    pltpu.async_copy(h.at[1], ib1, semi1)
    pltpu.make_async_copy(h.at[0], ib0, semi0).wait()
    pltpu.async_copy(x_hbm.at[ib0.at[1]], rows0, semr0)

    @pl.loop(0, (NCH - 1) // 2)
    def _(i):
        j = 2 * i
        pltpu.make_async_copy(x_hbm.at[ib0.at[1]], rows0, semr0).wait()
        pltpu.make_async_copy(h.at[j + 1], ib1, semi1).wait()
        pltpu.async_copy(x_hbm.at[ib1.at[1]], rows1, semr1)
        pltpu.sync_copy(rows0, acc.at[ib0.at[0]], add=True)
        pltpu.async_copy(h.at[j + 2], ib0, semi0)

        pltpu.make_async_copy(x_hbm.at[ib1.at[1]], rows1, semr1).wait()
        pltpu.make_async_copy(h.at[j + 2], ib0, semi0).wait()
        pltpu.async_copy(x_hbm.at[ib0.at[1]], rows0, semr0)
        pltpu.sync_copy(rows1, acc.at[ib1.at[0]], add=True)
        pltpu.async_copy(h.at[j + 3], ib1, semi1)

    pltpu.make_async_copy(x_hbm.at[ib0.at[1]], rows0, semr0).wait()
    pltpu.make_async_copy(h.at[NCH], ib1, semi1).wait()
    pltpu.sync_copy(rows0, acc.at[ib0.at[0]], add=True)

    plsc.subcore_barrier()
    sl = pl.ds(s * ROWS_PER_TILE, ROWS_PER_TILE)
    pltpu.sync_copy(acc.at[sl], out_hbm.at[c, sl])

    @pl.when(s == 0)
    def _():
        sl2 = pl.ds(NS * ROWS_PER_TILE, REM_ROWS)
        pltpu.sync_copy(acc.at[sl2], out_hbm.at[c, sl2])


_sc_scatter = functools.partial(
    pl.kernel,
    out_type=jax.ShapeDtypeStruct((NC, N_NODES, D), jnp.float32),
    mesh=plsc.VectorSubcoreMesh(core_axis_name="c", subcore_axis_name="s"),
    scratch_types=[
        pltpu.VMEM((2, CH), jnp.int32),
        pltpu.VMEM((2, CH), jnp.int32),
        pltpu.VMEM((CH, D), jnp.float32),
        pltpu.VMEM((CH, D), jnp.float32),
        pltpu.VMEM_SHARED((ACC_ROWS, D), jnp.float32),
        pltpu.SemaphoreType.DMA,
        pltpu.SemaphoreType.DMA,
        pltpu.SemaphoreType.DMA,
        pltpu.SemaphoreType.DMA,
    ],
)(_sc_body)


def _add_body(p_ref, q_ref, o_ref):
    o_ref[...] = p_ref[0] + q_ref[0]


def _tc_add(partials):
    blk = 1000
    return pl.pallas_call(
        _add_body,
        grid=(N_NODES // blk,),
        in_specs=[
            pl.BlockSpec((1, blk, D), lambda i: (0, i, 0)),
            pl.BlockSpec((1, blk, D), lambda i: (1, i, 0)),
        ],
        out_specs=pl.BlockSpec((blk, D), lambda i: (i, 0)),
        out_shape=jax.ShapeDtypeStruct((N_NODES, D), jnp.float32),
    )(partials, partials)


@jax.jit
def kernel(x, edge_index):
    pad = E_PAD - N_EDGES
    dst = jnp.pad(edge_index[0], (0, pad), constant_values=N_NODES)
    src = jnp.pad(edge_index[1], (0, pad), constant_values=0)
    pair = jnp.stack([dst.reshape(NT, NCH, CH), src.reshape(NT, NCH, CH)],
                     axis=2)
    pair = jnp.pad(pair, ((0, 0), (0, 2), (0, 0), (0, 0)))
    zeros = jnp.zeros((ROWS_PER_TILE, D), jnp.float32)
    partials = _sc_scatter(x, pair, zeros)
    return _tc_add(partials)


# speedup vs baseline: 1.7270x; 1.7270x over previous
import functools

import jax
import jax.numpy as jnp
from jax import lax
from jax.experimental import pallas as pl
from jax.experimental.pallas import tpu as pltpu
from jax.experimental.pallas import tpu_sc as plsc

N_NODES = 10000
N_EDGES = 320000
D = 128

NC = 2
NS = 16
CH = 80
EDGES_PER_TILE = N_EDGES // (NC * NS)
NCH = EDGES_PER_TILE // CH
ROWS_PER_TILE = 624
REM_ROWS = N_NODES - NS * ROWS_PER_TILE


def _sc_body(x_hbm, dst_hbm, src_hbm, zeros_hbm, out_hbm,
             idx_d, idx_s, rows0, rows1, acc, sem0, sem1):
    c = lax.axis_index("c")
    s = lax.axis_index("s")
    tid = c * NS + s

    pltpu.sync_copy(zeros_hbm.at[pl.ds(0, ROWS_PER_TILE)],
                    acc.at[pl.ds(s * ROWS_PER_TILE, ROWS_PER_TILE)])

    @pl.when(s == 0)
    def _():
        pltpu.sync_copy(zeros_hbm.at[pl.ds(0, REM_ROWS)],
                        acc.at[pl.ds(NS * ROWS_PER_TILE, REM_ROWS)])

    pltpu.sync_copy(dst_hbm.at[tid], idx_d)
    pltpu.sync_copy(src_hbm.at[tid], idx_s)
    plsc.subcore_barrier()

    pltpu.async_copy(x_hbm.at[idx_s.at[pl.ds(0, CH)]], rows0, sem0)

    @pl.loop(0, (NCH - 1) // 2)
    def _(i):
        j = 2 * i
        pltpu.make_async_copy(x_hbm.at[idx_s.at[pl.ds(0, CH)]], rows0,
                              sem0).wait()
        pltpu.async_copy(x_hbm.at[idx_s.at[pl.ds((j + 1) * CH, CH)]], rows1,
                         sem1)
        pltpu.sync_copy(rows0, acc.at[idx_d.at[j]], add=True)
        pltpu.make_async_copy(x_hbm.at[idx_s.at[pl.ds(0, CH)]], rows1,
                              sem1).wait()
        pltpu.async_copy(x_hbm.at[idx_s.at[pl.ds((j + 2) * CH, CH)]], rows0,
                         sem0)
        pltpu.sync_copy(rows1, acc.at[idx_d.at[j + 1]], add=True)

    pltpu.make_async_copy(x_hbm.at[idx_s.at[pl.ds(0, CH)]], rows0, sem0).wait()
    pltpu.sync_copy(rows0, acc.at[idx_d.at[NCH - 1]], add=True)

    plsc.subcore_barrier()
    sl = pl.ds(s * ROWS_PER_TILE, ROWS_PER_TILE)
    pltpu.sync_copy(acc.at[sl], out_hbm.at[c, sl])

    @pl.when(s == 0)
    def _():
        sl2 = pl.ds(NS * ROWS_PER_TILE, REM_ROWS)
        pltpu.sync_copy(acc.at[sl2], out_hbm.at[c, sl2])


_sc_scatter = functools.partial(
    pl.kernel,
    out_type=jax.ShapeDtypeStruct((NC, N_NODES, D), jnp.float32),
    mesh=plsc.VectorSubcoreMesh(core_axis_name="c", subcore_axis_name="s"),
    scratch_types=[
        pltpu.VMEM((NCH, CH), jnp.int32),
        pltpu.VMEM((EDGES_PER_TILE,), jnp.int32),
        pltpu.VMEM((CH, D), jnp.float32),
        pltpu.VMEM((CH, D), jnp.float32),
        pltpu.VMEM_SHARED((N_NODES, D), jnp.float32),
        pltpu.SemaphoreType.DMA,
        pltpu.SemaphoreType.DMA,
    ],
)(_sc_body)


def _add_body(p_ref, q_ref, o_ref):
    o_ref[...] = p_ref[0] + q_ref[0]


def _tc_add(partials):
    blk = 1000
    return pl.pallas_call(
        _add_body,
        grid=(N_NODES // blk,),
        in_specs=[
            pl.BlockSpec((1, blk, D), lambda i: (0, i, 0)),
            pl.BlockSpec((1, blk, D), lambda i: (1, i, 0)),
        ],
        out_specs=pl.BlockSpec((blk, D), lambda i: (i, 0)),
        out_shape=jax.ShapeDtypeStruct((N_NODES, D), jnp.float32),
    )(partials, partials)


@jax.jit
def kernel(x, edge_index):
    dst = edge_index[0].reshape(NC * NS, NCH, CH)
    src = edge_index[1].reshape(NC * NS, EDGES_PER_TILE)
    zeros = jnp.zeros((ROWS_PER_TILE, D), jnp.float32)
    partials = _sc_scatter(x, dst, src, zeros)
    return _tc_add(partials)


# speedup vs baseline: 1.8319x; 1.0607x over previous
import functools

import jax
import jax.numpy as jnp
from jax import lax
from jax.experimental import pallas as pl
from jax.experimental.pallas import tpu as pltpu
from jax.experimental.pallas import tpu_sc as plsc

N_NODES = 10000
N_EDGES = 320000
D = 128

NC = 2
NS = 16
CH = 80
EDGES_PER_TILE = N_EDGES // (NC * NS)
NCH = EDGES_PER_TILE // CH
ROWS_PER_TILE = 624
REM_ROWS = N_NODES - NS * ROWS_PER_TILE


def _sc_body(x_hbm, dst_hbm, src_hbm, zeros_hbm, out_hbm,
             idx_d, idx_s, rows0, rows1, acc, sem0, sem0b, sem1, sem1b):
    c = lax.axis_index("c")
    s = lax.axis_index("s")
    tid = c * NS + s

    pltpu.sync_copy(zeros_hbm.at[pl.ds(0, ROWS_PER_TILE)],
                    acc.at[pl.ds(s * ROWS_PER_TILE, ROWS_PER_TILE)])

    @pl.when(s == 0)
    def _():
        pltpu.sync_copy(zeros_hbm.at[pl.ds(0, REM_ROWS)],
                        acc.at[pl.ds(NS * ROWS_PER_TILE, REM_ROWS)])

    pltpu.sync_copy(dst_hbm.at[tid], idx_d)
    pltpu.sync_copy(src_hbm.at[tid], idx_s)
    plsc.subcore_barrier()

    H = CH // 2

    def _start_gather(j, rows, sa, sb):
        pltpu.async_copy(x_hbm.at[idx_s.at[pl.ds(j * CH, H)]],
                         rows.at[pl.ds(0, H)], sa)
        pltpu.async_copy(x_hbm.at[idx_s.at[pl.ds(j * CH + H, H)]],
                         rows.at[pl.ds(H, H)], sb)

    def _wait_gather(rows, sa, sb):
        pltpu.make_async_copy(x_hbm.at[idx_s.at[pl.ds(0, H)]],
                              rows.at[pl.ds(0, H)], sa).wait()
        pltpu.make_async_copy(x_hbm.at[idx_s.at[pl.ds(0, H)]],
                              rows.at[pl.ds(H, H)], sb).wait()

    _start_gather(0, rows0, sem0, sem0b)

    @pl.loop(0, (NCH - 1) // 2)
    def _(i):
        j = 2 * i
        _wait_gather(rows0, sem0, sem0b)
        _start_gather(j + 1, rows1, sem1, sem1b)
        pltpu.sync_copy(rows0, acc.at[idx_d.at[j]], add=True)
        _wait_gather(rows1, sem1, sem1b)
        _start_gather(j + 2, rows0, sem0, sem0b)
        pltpu.sync_copy(rows1, acc.at[idx_d.at[j + 1]], add=True)

    _wait_gather(rows0, sem0, sem0b)
    pltpu.sync_copy(rows0, acc.at[idx_d.at[NCH - 1]], add=True)

    plsc.subcore_barrier()
    sl = pl.ds(s * ROWS_PER_TILE, ROWS_PER_TILE)
    pltpu.sync_copy(acc.at[sl], out_hbm.at[c, sl])

    @pl.when(s == 0)
    def _():
        sl2 = pl.ds(NS * ROWS_PER_TILE, REM_ROWS)
        pltpu.sync_copy(acc.at[sl2], out_hbm.at[c, sl2])


_sc_scatter = functools.partial(
    pl.kernel,
    out_type=jax.ShapeDtypeStruct((NC, N_NODES, D), jnp.float32),
    mesh=plsc.VectorSubcoreMesh(core_axis_name="c", subcore_axis_name="s"),
    scratch_types=[
        pltpu.VMEM((NCH, CH), jnp.int32),
        pltpu.VMEM((EDGES_PER_TILE,), jnp.int32),
        pltpu.VMEM((CH, D), jnp.float32),
        pltpu.VMEM((CH, D), jnp.float32),
        pltpu.VMEM_SHARED((N_NODES, D), jnp.float32),
        pltpu.SemaphoreType.DMA,
        pltpu.SemaphoreType.DMA,
        pltpu.SemaphoreType.DMA,
        pltpu.SemaphoreType.DMA,
    ],
)(_sc_body)


def _add_body(p_ref, q_ref, o_ref):
    o_ref[...] = p_ref[0] + q_ref[0]


def _tc_add(partials):
    blk = 1000
    return pl.pallas_call(
        _add_body,
        grid=(N_NODES // blk,),
        in_specs=[
            pl.BlockSpec((1, blk, D), lambda i: (0, i, 0)),
            pl.BlockSpec((1, blk, D), lambda i: (1, i, 0)),
        ],
        out_specs=pl.BlockSpec((blk, D), lambda i: (i, 0)),
        out_shape=jax.ShapeDtypeStruct((N_NODES, D), jnp.float32),
    )(partials, partials)


@jax.jit
def kernel(x, edge_index):
    dst = edge_index[0].reshape(NC * NS, NCH, CH)
    src = edge_index[1].reshape(NC * NS, EDGES_PER_TILE)
    zeros = jnp.zeros((ROWS_PER_TILE, D), jnp.float32)
    partials = _sc_scatter(x, dst, src, zeros)
    return _tc_add(partials)
